# Initial kernel scaffold; baseline (speedup 1.0000x reference)
import jax
import jax.numpy as jnp
from jax.experimental import pallas as pl


def kernel(unit_embeddings, unit_confidence, unit_mask, boundary_mask):
    raise NotImplementedError("write your pallas kernel here")


# speedup vs baseline: 31.3754x; 31.3754x over previous
import functools

import jax
import jax.numpy as jnp
from jax import lax
from jax.experimental import pallas as pl
from jax.experimental.pallas import tpu as pltpu
from jax.experimental.pallas import tpu_sc as plsc

EPS_ = 1e-4
B_, J_, D_, L_ = 8, 2048, 512, 4096
CHUNK = 256
BCHUNK = 512
NW = 32
ROWS_PER_W = (B_ * L_) // NW
GCHUNK = 64
NGC = ROWS_PER_W // GCHUNK


def _ema_idx_kernel(emb_ref, conf_ref, mask_ref, bnd_ref, sm_ref, idx_ref):
    f32 = jnp.float32
    prec = lax.Precision.HIGHEST
    dn_t = (((1,), (1,)), ((), ()))
    dn_m = (((1,), (0,)), ((), ()))

    C = CHUNK
    ri = lax.broadcasted_iota(jnp.int32, (C, C), 0)
    ci = lax.broadcasted_iota(jnp.int32, (C, C), 1)
    tril = (ri >= ci).astype(f32)
    triu = (ri <= ci).astype(f32)
    eye = (ri == ci).astype(f32)

    def chunk(c, carry):
        sl = pl.ds(c * C, C)
        conf = conf_ref[0, 0:1, sl]
        msk = mask_ref[0, 0:1, sl]
        p = jnp.clip(conf, EPS_, 1.0 - EPS_)
        valid = msk > 0.5
        la = jnp.log(jnp.where(valid, 1.0 - p, 1.0))
        bcoef = jnp.where(valid, p, 0.0)
        cum_col = lax.dot_general(tril, la, dn_t, precision=prec,
                                  preferred_element_type=f32)
        cum_row = lax.dot_general(la, triu, dn_m, precision=prec,
                                  preferred_element_type=f32)
        m = jnp.where(ri >= ci, jnp.exp(cum_col - cum_row), 0.0)
        b_col = lax.dot_general(eye, bcoef, dn_t, precision=prec,
                                preferred_element_type=f32)
        emb = emb_ref[0, sl, :]
        sm = lax.dot_general(m, b_col * emb, dn_m, precision=prec,
                             preferred_element_type=f32)
        sm = sm + jnp.exp(cum_col) * carry
        sm_ref[0, sl, :] = sm
        return sm[C - 1:C, :]

    lax.fori_loop(0, J_ // C, chunk, jnp.zeros((1, D_), f32))

    CB = BCHUNK
    rb = lax.broadcasted_iota(jnp.int32, (CB, CB), 0)
    cb = lax.broadcasted_iota(jnp.int32, (CB, CB), 1)
    triu_b = (rb <= cb).astype(f32)
    boff = pl.program_id(0) * J_

    def bchunk(c, carry):
        sl = pl.ds(c * CB, CB)
        bnd = bnd_ref[0, 0:1, sl]
        cum = lax.dot_general(bnd, triu_b, dn_m, precision=prec,
                              preferred_element_type=f32) + carry
        idx = jnp.clip(cum - 1.0, 0.0, float(J_ - 1)).astype(jnp.int32)
        idx_ref[0, 0:1, sl] = idx + boff
        return cum[:, CB - 1:CB]

    lax.fori_loop(0, L_ // CB, bchunk, jnp.zeros((1, 1), f32))


def _ema_idx(emb, conf, msk, bnd):
    return pl.pallas_call(
        _ema_idx_kernel,
        grid=(B_,),
        in_specs=[
            pl.BlockSpec((1, J_, D_), lambda b: (b, 0, 0)),
            pl.BlockSpec((1, 1, J_), lambda b: (b, 0, 0)),
            pl.BlockSpec((1, 1, J_), lambda b: (b, 0, 0)),
            pl.BlockSpec((1, 1, L_), lambda b: (b, 0, 0)),
        ],
        out_specs=[
            pl.BlockSpec((1, J_, D_), lambda b: (b, 0, 0)),
            pl.BlockSpec((1, 1, L_), lambda b: (b, 0, 0)),
        ],
        out_shape=[
            jax.ShapeDtypeStruct((B_, J_, D_), jnp.float32),
            jax.ShapeDtypeStruct((B_, 1, L_), jnp.int32),
        ],
    )(emb, conf, msk, bnd)


def _sc_gather(table, idx3):
    mesh = plsc.VectorSubcoreMesh(core_axis_name="c", subcore_axis_name="s")

    @functools.partial(
        pl.kernel,
        mesh=mesh,
        out_type=jax.ShapeDtypeStruct((B_ * L_, D_), jnp.float32),
        scratch_types=[
            pltpu.VMEM((NGC, GCHUNK), jnp.int32),
            pltpu.VMEM((GCHUNK, D_), jnp.float32),
            pltpu.SemaphoreType.DMA,
        ],
    )
    def k(table_hbm, idx_hbm, out_hbm, idx_v, rows_v, sem):
        wid = lax.axis_index("s") * 2 + lax.axis_index("c")
        base = wid * ROWS_PER_W
        pltpu.sync_copy(idx_hbm.at[wid], idx_v)

        def body(c, carry):
            pltpu.async_copy(table_hbm.at[idx_v.at[c]], rows_v, sem).wait()
            pltpu.sync_copy(rows_v, out_hbm.at[pl.ds(base + c * GCHUNK, GCHUNK)])
            return carry

        lax.fori_loop(0, NGC, body, 0)

    return k(table, idx3)


def kernel(unit_embeddings, unit_confidence, unit_mask, boundary_mask):
    conf = unit_confidence.reshape(B_, 1, J_)
    msk = unit_mask.astype(jnp.float32).reshape(B_, 1, J_)
    bnd = boundary_mask.astype(jnp.float32).reshape(B_, 1, L_)
    smoothed, idx = _ema_idx(unit_embeddings, conf, msk, bnd)
    frames = _sc_gather(smoothed.reshape(B_ * J_, D_),
                        idx.reshape(NW, NGC, GCHUNK))
    return frames.reshape(B_, L_, D_)


# speedup vs baseline: 32.6801x; 1.0416x over previous
import functools

import jax
import jax.numpy as jnp
from jax import lax
from jax.experimental import pallas as pl
from jax.experimental.pallas import tpu as pltpu
from jax.experimental.pallas import tpu_sc as plsc

EPS_ = 1e-4
B_, J_, D_, L_ = 8, 2048, 512, 4096
CHUNK = 256
BCHUNK = 512
NW = 32
ROWS_PER_W = (B_ * L_) // NW
GCHUNK = 64
NGC = ROWS_PER_W // GCHUNK


def _ema_idx_kernel(emb_ref, conf_ref, mask_ref, bnd_ref, sm_ref, idx_ref):
    f32 = jnp.float32
    prec = lax.Precision.HIGHEST
    dn_t = (((1,), (1,)), ((), ()))
    dn_m = (((1,), (0,)), ((), ()))

    C = CHUNK
    ri = lax.broadcasted_iota(jnp.int32, (C, C), 0)
    ci = lax.broadcasted_iota(jnp.int32, (C, C), 1)
    tril = (ri >= ci).astype(f32)
    triu = (ri <= ci).astype(f32)
    eye = (ri == ci).astype(f32)

    def chunk(c, carry):
        sl = pl.ds(c * C, C)
        conf = conf_ref[0, 0:1, sl]
        msk = mask_ref[0, 0:1, sl]
        p = jnp.clip(conf, EPS_, 1.0 - EPS_)
        valid = msk > 0.5
        la = jnp.log(jnp.where(valid, 1.0 - p, 1.0))
        bcoef = jnp.where(valid, p, 0.0)
        cum_col = lax.dot_general(tril, la, dn_t, precision=prec,
                                  preferred_element_type=f32)
        cum_row = lax.dot_general(la, triu, dn_m, precision=prec,
                                  preferred_element_type=f32)
        m = jnp.where(ri >= ci, jnp.exp(cum_col - cum_row), 0.0)
        b_col = lax.dot_general(eye, bcoef, dn_t, precision=prec,
                                preferred_element_type=f32)
        emb = emb_ref[0, sl, :]
        sm = lax.dot_general(m, b_col * emb, dn_m, precision=prec,
                             preferred_element_type=f32)
        sm = sm + jnp.exp(cum_col) * carry
        sm_ref[0, sl, :] = sm
        return sm[C - 1:C, :]

    lax.fori_loop(0, J_ // C, chunk, jnp.zeros((1, D_), f32))

    CB = BCHUNK
    rb = lax.broadcasted_iota(jnp.int32, (CB, CB), 0)
    cb = lax.broadcasted_iota(jnp.int32, (CB, CB), 1)
    triu_b = (rb <= cb).astype(f32)
    boff = pl.program_id(0) * J_

    def bchunk(c, carry):
        sl = pl.ds(c * CB, CB)
        bnd = bnd_ref[0, 0:1, sl]
        cum = lax.dot_general(bnd, triu_b, dn_m, precision=prec,
                              preferred_element_type=f32) + carry
        idx = jnp.clip(cum - 1.0, 0.0, float(J_ - 1)).astype(jnp.int32)
        idx_ref[0, 0:1, sl] = idx + boff
        return cum[:, CB - 1:CB]

    lax.fori_loop(0, L_ // CB, bchunk, jnp.zeros((1, 1), f32))


def _ema_idx(emb, conf, msk, bnd):
    return pl.pallas_call(
        _ema_idx_kernel,
        grid=(B_,),
        in_specs=[
            pl.BlockSpec((1, J_, D_), lambda b: (b, 0, 0)),
            pl.BlockSpec((1, 1, J_), lambda b: (b, 0, 0)),
            pl.BlockSpec((1, 1, J_), lambda b: (b, 0, 0)),
            pl.BlockSpec((1, 1, L_), lambda b: (b, 0, 0)),
        ],
        out_specs=[
            pl.BlockSpec((1, J_, D_), lambda b: (b, 0, 0)),
            pl.BlockSpec((1, 1, L_), lambda b: (b, 0, 0)),
        ],
        out_shape=[
            jax.ShapeDtypeStruct((B_, J_, D_), jnp.float32),
            jax.ShapeDtypeStruct((B_, 1, L_), jnp.int32),
        ],
    )(emb, conf, msk, bnd)


def _sc_gather(table, idx3):
    mesh = plsc.VectorSubcoreMesh(core_axis_name="c", subcore_axis_name="s")

    @functools.partial(
        pl.kernel,
        mesh=mesh,
        out_type=jax.ShapeDtypeStruct((B_ * L_, D_), jnp.float32),
        scratch_types=[
            pltpu.VMEM((NGC, GCHUNK), jnp.int32),
            pltpu.VMEM((2, GCHUNK, D_), jnp.float32),
            pltpu.SemaphoreType.DMA,
        ],
    )
    def k(table_hbm, idx_hbm, out_hbm, idx_v, rows_v, sem):
        wid = lax.axis_index("s") * 2 + lax.axis_index("c")
        base = wid * ROWS_PER_W
        pltpu.sync_copy(idx_hbm.at[wid], idx_v)
        pltpu.async_copy(table_hbm.at[idx_v.at[0]], rows_v.at[0], sem)

        def body(c, carry):
            p = lax.rem(c, 2)
            pltpu.make_async_copy(table_hbm.at[idx_v.at[c]],
                                  rows_v.at[p], sem).wait()

            @pl.when(c + 1 < NGC)
            def _():
                pltpu.async_copy(table_hbm.at[idx_v.at[c + 1]],
                                 rows_v.at[1 - p], sem)

            pltpu.sync_copy(rows_v.at[p],
                            out_hbm.at[pl.ds(base + c * GCHUNK, GCHUNK)])
            return carry

        lax.fori_loop(0, NGC, body, 0)

    return k(table, idx3)


def kernel(unit_embeddings, unit_confidence, unit_mask, boundary_mask):
    conf = unit_confidence.reshape(B_, 1, J_)
    msk = unit_mask.astype(jnp.float32).reshape(B_, 1, J_)
    bnd = boundary_mask.astype(jnp.float32).reshape(B_, 1, L_)
    smoothed, idx = _ema_idx(unit_embeddings, conf, msk, bnd)
    frames = _sc_gather(smoothed.reshape(B_ * J_, D_),
                        idx.reshape(NW, NGC, GCHUNK))
    return frames.reshape(B_, L_, D_)


# speedup vs baseline: 50.0503x; 1.5315x over previous
import functools

import jax
import jax.numpy as jnp
from jax import lax
from jax.experimental import pallas as pl
from jax.experimental.pallas import tpu as pltpu
from jax.experimental.pallas import tpu_sc as plsc

EPS_ = 1e-4
B_, J_, D_, L_ = 8, 2048, 512, 4096
CHUNK = 256
NCH_ = J_ // CHUNK
BCHUNK = 512
NBC_ = L_ // BCHUNK
NW = 32
ROWS_PER_W = (B_ * L_) // NW
GCHUNK = 64
NGC = ROWS_PER_W // GCHUNK


def _ema_idx_kernel(emb_ref, conf_ref, mask_ref, bnd_ref, sm_ref, idx_ref):
    f32 = jnp.float32
    dn_m = (((1,), (0,)), ((), ()))
    dn_t = (((1,), (1,)), ((), ()))
    HI = lax.Precision.HIGHEST
    DF = lax.Precision.DEFAULT

    C = CHUNK
    ri = lax.broadcasted_iota(jnp.int32, (C, C), 0)
    ci = lax.broadcasted_iota(jnp.int32, (C, C), 1)
    triu = (ri <= ci).astype(f32)
    eye = (ri == ci).astype(f32)

    conf = conf_ref[0]
    msk = mask_ref[0]
    p = jnp.clip(conf, EPS_, 1.0 - EPS_)
    valid = msk > 0.5
    la = jnp.log(jnp.where(valid, 1.0 - p, 1.0))
    bcoef = jnp.where(valid, p, 0.0)
    cum = lax.dot_general(la, triu, dn_m, precision=HI,
                          preferred_element_type=f32)
    cumt = lax.dot_general(eye, cum, dn_t, precision=HI,
                           preferred_element_type=f32)
    bt = lax.dot_general(eye, bcoef, dn_t, precision=HI,
                         preferred_element_type=f32)
    ecolt = jnp.exp(cumt)

    carry = jnp.zeros((1, D_), f32)
    for c in range(NCH_):
        m = jnp.where(ri >= ci, jnp.exp(cumt[:, c:c + 1] - cum[c:c + 1, :]),
                      0.0)
        bmat = bt[:, c:c + 1] * emb_ref[0, c * C:(c + 1) * C, :]
        sm = lax.dot_general(m, bmat, dn_m, precision=DF,
                             preferred_element_type=f32)
        sm = sm + ecolt[:, c:c + 1] * carry
        sm_ref[0, c * C:(c + 1) * C, :] = sm
        carry = sm[C - 1:C, :]

    CB = BCHUNK
    rb = lax.broadcasted_iota(jnp.int32, (CB, CB), 0)
    cbi = lax.broadcasted_iota(jnp.int32, (CB, CB), 1)
    triu_b = (rb <= cbi).astype(f32)
    r8 = lax.broadcasted_iota(jnp.int32, (NBC_, NBC_), 0)
    c8 = lax.broadcasted_iota(jnp.int32, (NBC_, NBC_), 1)
    trilx = (r8 > c8).astype(f32)

    bnd = bnd_ref[0]
    cumb = lax.dot_general(bnd, triu_b, dn_m, precision=DF,
                           preferred_element_type=f32)
    tot = cumb[:, CB - 1:CB]
    pre = lax.dot_general(trilx, tot, dn_m, precision=lax.Precision.HIGHEST,
                          preferred_element_type=f32)
    cum_all = cumb + pre
    idx = jnp.clip(cum_all - 1.0, 0.0, float(J_ - 1)).astype(jnp.int32)
    idx_ref[0] = idx + pl.program_id(0) * J_


def _ema_idx(emb, conf, msk, bnd):
    return pl.pallas_call(
        _ema_idx_kernel,
        grid=(B_,),
        in_specs=[
            pl.BlockSpec((1, J_, D_), lambda b: (b, 0, 0)),
            pl.BlockSpec((1, NCH_, CHUNK), lambda b: (b, 0, 0)),
            pl.BlockSpec((1, NCH_, CHUNK), lambda b: (b, 0, 0)),
            pl.BlockSpec((1, NBC_, BCHUNK), lambda b: (b, 0, 0)),
        ],
        out_specs=[
            pl.BlockSpec((1, J_, D_), lambda b: (b, 0, 0)),
            pl.BlockSpec((1, NBC_, BCHUNK), lambda b: (b, 0, 0)),
        ],
        out_shape=[
            jax.ShapeDtypeStruct((B_, J_, D_), jnp.float32),
            jax.ShapeDtypeStruct((B_, NBC_, BCHUNK), jnp.int32),
        ],
    )(emb, conf, msk, bnd)


def _sc_gather(table, idx3):
    mesh = plsc.VectorSubcoreMesh(core_axis_name="c", subcore_axis_name="s")

    @functools.partial(
        pl.kernel,
        mesh=mesh,
        out_type=jax.ShapeDtypeStruct((B_ * L_, D_), jnp.float32),
        scratch_types=[
            pltpu.VMEM((NGC, GCHUNK), jnp.int32),
            pltpu.VMEM((2, GCHUNK, D_), jnp.float32),
            pltpu.SemaphoreType.DMA,
        ],
    )
    def k(table_hbm, idx_hbm, out_hbm, idx_v, rows_v, sem):
        wid = lax.axis_index("s") * 2 + lax.axis_index("c")
        base = wid * ROWS_PER_W
        pltpu.sync_copy(idx_hbm.at[wid], idx_v)
        pltpu.async_copy(table_hbm.at[idx_v.at[0]], rows_v.at[0], sem)

        def body(c, carry):
            p = lax.rem(c, 2)
            pltpu.make_async_copy(table_hbm.at[idx_v.at[c]],
                                  rows_v.at[p], sem).wait()

            @pl.when(c + 1 < NGC)
            def _():
                pltpu.async_copy(table_hbm.at[idx_v.at[c + 1]],
                                 rows_v.at[1 - p], sem)

            pltpu.sync_copy(rows_v.at[p],
                            out_hbm.at[pl.ds(base + c * GCHUNK, GCHUNK)])
            return carry

        lax.fori_loop(0, NGC, body, 0)

    return k(table, idx3)


def kernel(unit_embeddings, unit_confidence, unit_mask, boundary_mask):
    conf = unit_confidence.reshape(B_, NCH_, CHUNK)
    msk = unit_mask.astype(jnp.float32).reshape(B_, NCH_, CHUNK)
    bnd = boundary_mask.astype(jnp.float32).reshape(B_, NBC_, BCHUNK)
    smoothed, idx = _ema_idx(unit_embeddings, conf, msk, bnd)
    frames = _sc_gather(smoothed.reshape(B_ * J_, D_),
                        idx.reshape(NW, NGC, GCHUNK))
    return frames.reshape(B_, L_, D_)
